# baseline (speedup 1.0000x reference)
import jax
import jax.numpy as jnp
from jax import lax
from jax.experimental import pallas as pl
from jax.experimental.pallas import tpu as pltpu
from jax.experimental.pallas import tpu_sc as plsc

NUM_CORES = 2
NUM_SUBCORES = 16
NUM_WORKERS = NUM_CORES * NUM_SUBCORES
CHUNK = 128
EMB = 128


NBUF = 6
LOOKAHEAD = 3


def _gather_body(idx_hbm, table_hbm, out_hbm, idx_v, rows_v, gsem, wsem):
    wid = lax.axis_index("s") * NUM_CORES + lax.axis_index("c")
    k = idx_hbm.shape[1]
    base = wid * (k * CHUNK)
    pltpu.sync_copy(idx_hbm.at[wid], idx_v)

    def issue_gather(g, buf):
        return pltpu.async_copy(
            table_hbm.at[idx_v.at[g]], rows_v.at[buf], gsem.at[buf]
        )

    def issue_writeback(g, buf):
        return pltpu.async_copy(
            rows_v.at[buf],
            out_hbm.at[pl.ds(base + g * CHUNK, CHUNK)],
            wsem.at[buf],
        )

    gd = [None] * NBUF
    wd = [None] * NBUF
    for g in range(min(LOOKAHEAD, k)):
        gd[g % NBUF] = issue_gather(g, g % NBUF)
    for g in range(k):
        buf = g % NBUF
        gd[buf].wait()
        wd[buf] = issue_writeback(g, buf)
        nxt = g + LOOKAHEAD
        if nxt < k:
            nb = nxt % NBUF
            if wd[nb] is not None:
                wd[nb].wait()
                wd[nb] = None
            gd[nb] = issue_gather(nxt, nb)
    for d in wd:
        if d is not None:
            d.wait()


@jax.jit
def kernel(token_ids, weight):
    b, s = token_ids.shape
    total = b * s
    k = total // (NUM_WORKERS * CHUNK)
    idx = token_ids.astype(jnp.int32).reshape(NUM_WORKERS, k, CHUNK)
    mesh = plsc.VectorSubcoreMesh(core_axis_name="c", subcore_axis_name="s")
    out = pl.kernel(
        _gather_body,
        out_type=jax.ShapeDtypeStruct((total, EMB), jnp.float32),
        mesh=mesh,
        scratch_types=[
            pltpu.VMEM((k, CHUNK), jnp.int32),
            pltpu.VMEM((NBUF, CHUNK, EMB), jnp.float32),
            pltpu.SemaphoreType.DMA((NBUF,)),
            pltpu.SemaphoreType.DMA((NBUF,)),
        ],
    )(idx, weight)
    return out.reshape(b, s, EMB)


# speedup vs baseline: 1.7263x; 1.7263x over previous
import jax
import jax.numpy as jnp
from jax import lax
from jax.experimental import pallas as pl
from jax.experimental.pallas import tpu as pltpu
from jax.experimental.pallas import tpu_sc as plsc

NUM_CORES = 2
NUM_SUBCORES = 16
NUM_WORKERS = NUM_CORES * NUM_SUBCORES
EMB = 128
TGRP = 4
NBUF = 4
LOOKAHEAD = 2


def _gather_body(idx_hbm, table_hbm, out_hbm, idx_v, rows_v, gsem, wsem):
    wid = lax.axis_index("s") * NUM_CORES + lax.axis_index("c")
    toks = idx_hbm.shape[0] // NUM_WORKERS
    seq = idx_hbm.shape[1]
    ngrp = toks // TGRP
    tok_base = wid * toks
    pltpu.sync_copy(idx_hbm.at[pl.ds(tok_base, toks)], idx_v)

    def issue_gather(g, buf):
        descs = []
        for t in range(TGRP):
            descs.append(
                pltpu.async_copy(
                    table_hbm.at[idx_v.at[g * TGRP + t]],
                    rows_v.at[buf].at[t],
                    gsem.at[buf],
                )
            )
        return descs

    def issue_writeback(g, buf):
        return pltpu.async_copy(
            rows_v.at[buf],
            out_hbm.at[pl.ds(tok_base + g * TGRP, TGRP)],
            wsem.at[buf],
        )

    gd = [None] * NBUF
    wd = [None] * NBUF
    for g in range(min(LOOKAHEAD, ngrp)):
        gd[g % NBUF] = issue_gather(g, g % NBUF)
    for g in range(ngrp):
        buf = g % NBUF
        for d in gd[buf]:
            d.wait()
        wd[buf] = issue_writeback(g, buf)
        nxt = g + LOOKAHEAD
        if nxt < ngrp:
            nb = nxt % NBUF
            if wd[nb] is not None:
                wd[nb].wait()
                wd[nb] = None
            gd[nb] = issue_gather(nxt, nb)
    for d in wd:
        if d is not None:
            d.wait()


@jax.jit
def kernel(token_ids, weight):
    b, s = token_ids.shape
    idx = token_ids.astype(jnp.int32)
    mesh = plsc.VectorSubcoreMesh(core_axis_name="c", subcore_axis_name="s")
    toks = b // NUM_WORKERS
    out = pl.kernel(
        _gather_body,
        out_type=jax.ShapeDtypeStruct((b, s, EMB), jnp.float32),
        mesh=mesh,
        scratch_types=[
            pltpu.VMEM((toks, s), jnp.int32),
            pltpu.VMEM((NBUF, TGRP, s, EMB), jnp.float32),
            pltpu.SemaphoreType.DMA((NBUF,)),
            pltpu.SemaphoreType.DMA((NBUF,)),
        ],
    )(idx, weight)
    return out


# speedup vs baseline: 1.7269x; 1.0003x over previous
import jax
import jax.numpy as jnp
from jax import lax
from jax.experimental import pallas as pl
from jax.experimental.pallas import tpu as pltpu
from jax.experimental.pallas import tpu_sc as plsc

NUM_CORES = 2
NUM_SUBCORES = 16
NUM_WORKERS = NUM_CORES * NUM_SUBCORES
EMB = 128
TGRP = 4
NBUF = 4
LOOKAHEAD = 2


def _gather_body(idx_hbm, table_hbm, out_hbm, idx_v, rows_v, gsem, wsem):
    wid = lax.axis_index("s") * NUM_CORES + lax.axis_index("c")
    toks = idx_hbm.shape[0] // NUM_WORKERS
    seq_pad = idx_hbm.shape[1]
    seq = out_hbm.shape[1]
    ngrp = toks // TGRP
    tok_base = wid * toks
    pltpu.sync_copy(idx_hbm.at[pl.ds(tok_base, toks)], idx_v)

    def issue_gather(g, buf):
        return [
            pltpu.async_copy(
                table_hbm.at[idx_v.at[g * TGRP + t, pl.ds(0, seq)]],
                rows_v.at[buf].at[t],
                gsem.at[buf],
            )
            for t in range(TGRP)
        ]

    def issue_writeback(g, buf):
        return pltpu.async_copy(
            rows_v.at[buf],
            out_hbm.at[pl.ds(tok_base + g * TGRP, TGRP)],
            wsem.at[buf],
        )

    gd = [None] * NBUF
    wd = [None] * NBUF
    for g in range(min(LOOKAHEAD, ngrp)):
        gd[g % NBUF] = issue_gather(g, g % NBUF)
    for g in range(ngrp):
        buf = g % NBUF
        for d in gd[buf]:
            d.wait()
        wd[buf] = issue_writeback(g, buf)
        nxt = g + LOOKAHEAD
        if nxt < ngrp:
            nb = nxt % NBUF
            if wd[nb] is not None:
                wd[nb].wait()
                wd[nb] = None
            gd[nb] = issue_gather(nxt, nb)
    for d in wd:
        if d is not None:
            d.wait()


@jax.jit
def kernel(token_ids, weight):
    b, s = token_ids.shape
    s_pad = (s + 7) // 8 * 8
    idx = token_ids.astype(jnp.int32)
    if s_pad != s:
        idx = jnp.pad(idx, ((0, 0), (0, s_pad - s)))
    mesh = plsc.VectorSubcoreMesh(core_axis_name="c", subcore_axis_name="s")
    toks = b // NUM_WORKERS
    out = pl.kernel(
        _gather_body,
        out_type=jax.ShapeDtypeStruct((b, s, EMB), jnp.float32),
        mesh=mesh,
        scratch_types=[
            pltpu.VMEM((toks, s_pad), jnp.int32),
            pltpu.VMEM((NBUF, TGRP, s, EMB), jnp.float32),
            pltpu.SemaphoreType.DMA((NBUF,)),
            pltpu.SemaphoreType.DMA((NBUF,)),
        ],
    )(idx, weight)
    return out


# speedup vs baseline: 1.7305x; 1.0021x over previous
import jax
import jax.numpy as jnp
from jax import lax
from jax.experimental import pallas as pl
from jax.experimental.pallas import tpu as pltpu
from jax.experimental.pallas import tpu_sc as plsc

NUM_CORES = 2
NUM_SUBCORES = 16
NUM_WORKERS = NUM_CORES * NUM_SUBCORES
EMB = 128
TGRP = 4
NBUF = 4
LOOKAHEAD = 2


def _gather_body(idx_hbm, table_hbm, out_hbm, idx_v, rows_v, gsem, wsem):
    wid = lax.axis_index("s") * NUM_CORES + lax.axis_index("c")
    ntok, seq = out_hbm.shape[0], out_hbm.shape[1]
    toks = ntok // NUM_WORKERS
    spad = idx_hbm.shape[0] // ntok
    ngrp = toks // TGRP
    tok_base = wid * toks
    pltpu.sync_copy(idx_hbm.at[pl.ds(tok_base * spad, toks * spad)], idx_v)

    def issue_gather(g, buf):
        return [
            pltpu.async_copy(
                table_hbm.at[idx_v.at[pl.ds((g * TGRP + t) * spad, seq)]],
                rows_v.at[buf].at[t],
                gsem.at[buf],
            )
            for t in range(TGRP)
        ]

    def issue_writeback(g, buf):
        return pltpu.async_copy(
            rows_v.at[buf],
            out_hbm.at[pl.ds(tok_base + g * TGRP, TGRP)],
            wsem.at[buf],
        )

    gd = [None] * NBUF
    wd = [None] * NBUF
    for g in range(min(LOOKAHEAD, ngrp)):
        gd[g % NBUF] = issue_gather(g, g % NBUF)
    for g in range(ngrp):
        buf = g % NBUF
        for d in gd[buf]:
            d.wait()
        wd[buf] = issue_writeback(g, buf)
        nxt = g + LOOKAHEAD
        if nxt < ngrp:
            nb = nxt % NBUF
            if wd[nb] is not None:
                wd[nb].wait()
                wd[nb] = None
            gd[nb] = issue_gather(nxt, nb)
    for d in wd:
        if d is not None:
            d.wait()


@jax.jit
def kernel(token_ids, weight):
    b, s = token_ids.shape
    s_pad = (s + 7) // 8 * 8
    idx = token_ids.astype(jnp.int32)
    if s_pad != s:
        idx = jnp.pad(idx, ((0, 0), (0, s_pad - s)))
    idx = idx.reshape(b * s_pad)
    mesh = plsc.VectorSubcoreMesh(core_axis_name="c", subcore_axis_name="s")
    toks = b // NUM_WORKERS
    out = pl.kernel(
        _gather_body,
        out_type=jax.ShapeDtypeStruct((b, s, EMB), jnp.float32),
        mesh=mesh,
        compiler_params=pltpu.CompilerParams(use_tc_tiling_on_sc=True),
        scratch_types=[
            pltpu.VMEM((toks * s_pad,), jnp.int32),
            pltpu.VMEM((NBUF, TGRP, s, EMB), jnp.float32),
            pltpu.SemaphoreType.DMA((NBUF,)),
            pltpu.SemaphoreType.DMA((NBUF,)),
        ],
    )(idx, weight)
    return out


# speedup vs baseline: 3.1501x; 1.8204x over previous
import jax
import jax.numpy as jnp
from jax import lax
from jax.experimental import pallas as pl
from jax.experimental.pallas import tpu as pltpu
from jax.experimental.pallas import tpu_sc as plsc

NUM_CORES = 2
NUM_SUBCORES = 16
NUM_WORKERS = NUM_CORES * NUM_SUBCORES
EMB = 128
NBUF = 7
LOOKAHEAD = 4


def _gather_body(idx_hbm, table_hbm, out_hbm, idx_v, rows_v, gsem, wsem, isem):
    wid = lax.axis_index("s") * NUM_CORES + lax.axis_index("c")
    seq, batch = out_hbm.shape[0], out_hbm.shape[1]
    bpw = batch // NUM_WORKERS
    b_base = wid * bpw
    head = min(8, seq)
    pltpu.sync_copy(
        idx_hbm.at[pl.ds(0, head), pl.ds(b_base, bpw)],
        idx_v.at[pl.ds(0, head)],
    )
    idx_rest = None
    if head < seq:
        idx_rest = pltpu.async_copy(
            idx_hbm.at[pl.ds(head, seq - head), pl.ds(b_base, bpw)],
            idx_v.at[pl.ds(head, seq - head)],
            isem,
        )

    def issue_gather(s, buf):
        return pltpu.async_copy(
            table_hbm.at[idx_v.at[s]], rows_v.at[buf], gsem.at[buf]
        )

    def issue_writeback(s, buf):
        return pltpu.async_copy(
            rows_v.at[buf],
            out_hbm.at[s, pl.ds(b_base, bpw)],
            wsem.at[buf],
        )

    gd = [None] * NBUF
    wd = [None] * NBUF
    for s in range(min(LOOKAHEAD, seq)):
        gd[s % NBUF] = issue_gather(s, s % NBUF)
    for s in range(seq):
        buf = s % NBUF
        gd[buf].wait()
        wd[buf] = issue_writeback(s, buf)
        nxt = s + LOOKAHEAD
        if nxt < seq:
            if nxt >= head and idx_rest is not None:
                idx_rest.wait()
                idx_rest = None
            nb = nxt % NBUF
            if wd[nb] is not None:
                wd[nb].wait()
                wd[nb] = None
            gd[nb] = issue_gather(nxt, nb)
    for d in wd:
        if d is not None:
            d.wait()


@jax.jit
def kernel(token_ids, weight):
    b, s = token_ids.shape
    idx = token_ids.astype(jnp.int32).T
    mesh = plsc.VectorSubcoreMesh(core_axis_name="c", subcore_axis_name="s")
    bpw = b // NUM_WORKERS
    out = pl.kernel(
        _gather_body,
        out_type=jax.ShapeDtypeStruct((s, b, EMB), jnp.float32),
        mesh=mesh,
        scratch_types=[
            pltpu.VMEM((s, bpw), jnp.int32),
            pltpu.VMEM((NBUF, bpw, EMB), jnp.float32),
            pltpu.SemaphoreType.DMA((NBUF,)),
            pltpu.SemaphoreType.DMA((NBUF,)),
            pltpu.SemaphoreType.DMA,
        ],
    )(idx, weight)
    return out.transpose(1, 0, 2)
